# baseline (speedup 1.0000x reference)
import jax
import jax.numpy as jnp
from jax.experimental import pallas as pl
from jax.experimental.pallas import tpu as pltpu

_MOMENTUM = 0.01
_K = 4

_SIZES = [64, 128, 256, 512, 1024] + [1792] * 53 + [1056] + [1024, 512, 256, 128, 64]
_OFFS = [sum(_SIZES[:i]) for i in range(len(_SIZES))]
_MAXC = max(_SIZES)


def _body(x_hbm, tgt_ref, pen_ref, o_hbm, xbuf, obuf, insem, outsem):
    n = len(_SIZES)
    t = (1.0 - _MOMENTUM) * jnp.mean(tgt_ref[...])
    p = pen_ref[...]

    def in_copy(i, slot):
        return pltpu.make_async_copy(
            x_hbm.at[pl.ds(_OFFS[i], _SIZES[i]), :],
            xbuf.at[slot, pl.ds(0, _SIZES[i]), :],
            insem.at[slot],
        )

    def out_copy(i, slot):
        return pltpu.make_async_copy(
            obuf.at[slot, pl.ds(0, _SIZES[i]), :],
            o_hbm.at[pl.ds(_OFFS[i], _SIZES[i]), :],
            outsem.at[slot],
        )

    for i in range(_K):
        in_copy(i, i).start()

    for i in range(n):
        slot = i % _K
        in_copy(i, slot).wait()
        if i >= _K:
            out_copy(i - _K, slot).wait()
        x = xbuf[slot, pl.ds(0, _SIZES[i]), :]
        obuf[slot, pl.ds(0, _SIZES[i]), :] = jnp.where(x > p, x * (t + x), x)
        out_copy(i, slot).start()
        if i + _K < n:
            in_copy(i + _K, slot).start()

    for i in range(n - _K, n):
        out_copy(i, i % _K).wait()


def kernel(cosine_theta, cosine_theta_target, penalty_cosine_theta):
    B, C = cosine_theta.shape
    xt = cosine_theta.T
    tgt = cosine_theta_target.T
    pen = penalty_cosine_theta.T
    out_t = pl.pallas_call(
        _body,
        in_specs=[
            pl.BlockSpec(memory_space=pl.ANY),
            pl.BlockSpec(memory_space=pltpu.VMEM),
            pl.BlockSpec(memory_space=pltpu.VMEM),
        ],
        out_specs=pl.BlockSpec(memory_space=pl.ANY),
        out_shape=jax.ShapeDtypeStruct((C, B), cosine_theta.dtype),
        scratch_shapes=[
            pltpu.VMEM((_K, _MAXC, B), jnp.float32),
            pltpu.VMEM((_K, _MAXC, B), jnp.float32),
            pltpu.SemaphoreType.DMA((_K,)),
            pltpu.SemaphoreType.DMA((_K,)),
        ],
    )(xt, tgt, pen)
    return out_t.T
